# baseline (speedup 1.0000x reference)
import functools

import jax
import jax.numpy as jnp
from jax import lax
from jax.experimental import pallas as pl
from jax.experimental.pallas import tpu as pltpu
from jax.experimental.pallas import tpu_sc as plsc

_D = 128
_L = 80
_K = 5
_NEG = 5
_SLOTS = 2 * _K
_NNEG = _L * _SLOTS * _NEG
_NROWS = _L + _NNEG
_NW = 32
_B = 4096
_BPW = _B // _NW


def _sc_gather(table, mp, neg):
    mesh = plsc.VectorSubcoreMesh(core_axis_name="c", subcore_axis_name="s")

    @functools.partial(
        pl.kernel,
        mesh=mesh,
        out_type=jax.ShapeDtypeStruct((_B, _D), jnp.float32),
        scratch_types=[
            pltpu.VMEM((_BPW,), jnp.int32),
            pltpu.VMEM((_BPW, _D), jnp.float32),
            pltpu.SemaphoreType.DMA,
        ],
    )
    def gather_kernel(table_hbm, mp_hbm, neg_hbm, out_hbm, idx_v, rows_v, sem):
        wid = lax.axis_index("s") * 2 + lax.axis_index("c")
        base = wid * _BPW

        @pl.when(wid == 0)
        def _():
            pltpu.sync_copy(mp_hbm, idx_v.at[pl.ds(0, _L)])
            pltpu.sync_copy(neg_hbm.at[pl.ds(0, _BPW - _L)],
                            idx_v.at[pl.ds(_L, _BPW - _L)])

        @pl.when((wid > 0) & (wid < _NW - 1))
        def _():
            pltpu.sync_copy(neg_hbm.at[pl.ds(base - _L, _BPW)], idx_v)

        @pl.when(wid == _NW - 1)
        def _():
            tail = _NNEG - ((_NW - 1) * _BPW - _L)
            pltpu.sync_copy(neg_hbm.at[pl.ds(_NNEG - tail, tail)],
                            idx_v.at[pl.ds(0, tail)])
            for t in range(tail, _BPW, 16):
                idx_v[pl.ds(t, 16)] = jnp.zeros((16,), jnp.int32)

        pltpu.async_copy(table_hbm.at[idx_v], rows_v, sem).wait()
        pltpu.sync_copy(rows_v, out_hbm.at[pl.ds(base, _BPW)])

    return gather_kernel(table, mp, neg)


def _tc_loss_kernel(rows_ref, out_ref):
    walk = rows_ref[0:_L, :]
    negs = rows_ref[_L:_L + _NNEG, :]

    gram = lax.dot_general(walk, walk, (((1,), (1,)), ((), ())),
                           preferred_element_type=jnp.float32)
    ii = lax.broadcasted_iota(jnp.int32, (_L, _L), 0)
    jj = lax.broadcasted_iota(jnp.int32, (_L, _L), 1)
    dij = jnp.abs(ii - jj)
    mask_pos = (dij >= 1) & (dij <= _K)

    def softplus(x):
        return jnp.maximum(x, 0.0) + jnp.log1p(jnp.exp(-jnp.abs(x)))

    per_owner = _SLOTS * _NEG
    gsz = 400
    gown = gsz // per_owner
    neg_loss = jnp.float32(0.0)
    for g in range(_NNEG // gsz):
        ns = negs[g * gsz:(g + 1) * gsz, :]
        ws = walk[g * gown:(g + 1) * gown, :]
        ndg = lax.dot_general(ns, ws, (((1,), (1,)), ((), ())),
                              preferred_element_type=jnp.float32)
        lr = lax.broadcasted_iota(jnp.int32, (gsz, gown), 0)
        cc = lax.broadcasted_iota(jnp.int32, (gsz, gown), 1)
        oloc = lr // per_owner
        owner = oloc + g * gown
        slot = (lr % per_owner) // _NEG
        wsize = (jnp.minimum(owner + _K, _L - 1)
                 - jnp.maximum(owner - _K, 0))
        mask = (cc == oloc) & (slot < wsize)
        neg_loss = neg_loss + jnp.sum(jnp.where(mask, softplus(ndg), 0.0))

    pos_loss = jnp.sum(jnp.where(mask_pos, softplus(-gram), 0.0))
    n_pairs = jnp.sum(mask_pos.astype(jnp.float32))
    out_ref[0, 0] = (pos_loss + neg_loss) / n_pairs


def kernel(MP, neg_samples, X):
    mp = MP.astype(jnp.int32)
    neg = neg_samples.astype(jnp.int32).reshape(-1)
    rows = _sc_gather(X, mp, neg)
    loss = pl.pallas_call(
        _tc_loss_kernel,
        out_shape=jax.ShapeDtypeStruct((1, 1), jnp.float32),
        out_specs=pl.BlockSpec(memory_space=pltpu.SMEM),
    )(rows)
    return loss[0, 0]
